# baseline (speedup 1.0000x reference)
import functools

import jax
import jax.numpy as jnp
from jax import lax
from jax.experimental import pallas as pl
from jax.experimental.pallas import tpu as pltpu
from jax.experimental.pallas import tpu_sc as plsc

EMB = 128
B = 4096
L = 20
N_BIG = 3
N_SMALL = 16
N_SEQ = 5
AG = 3
SC_FEAT = N_BIG + N_SEQ

NC = 2
NS = 16
NW = NC * NS
BW = B // NW
CR = 16
NCH = BW // CR
CI = CR * L
GN = 4
GI = CI // GN


def _sc_body(*refs):
    btab = refs[0:N_BIG]
    bidx = refs[N_BIG:2 * N_BIG]
    qtab = refs[6:6 + N_SEQ]
    qidx = refs[11:11 + N_SEQ]
    out = refs[16]
    (idxv, rowsA, rowsB, gidx, gidx1A, gidx1B, seqA, seqB, pool, w0v,
     sg0, sg1, sw0, sw1, qi0, qi1, qg0, qg1, qw0, qw1) = refs[17:]
    qi = (qi0, qi1)
    qg = (qg0, qg1)
    qw = (qw0, qw1)
    seqbuf = (seqA, seqB)
    gidx1 = (gidx1A, gidx1B)

    wid = lax.axis_index("s") * NC + lax.axis_index("c")
    base = wid * BW

    pltpu.sync_copy(refs[6 + AG].at[pl.ds(0, 1)], w0v)

    pltpu.sync_copy(bidx[0].at[pl.ds(base, BW)], idxv.at[0])
    g0 = pltpu.make_async_copy(btab[0].at[idxv.at[0]], rowsA, sg0)
    g0.start()
    pltpu.sync_copy(bidx[1].at[pl.ds(base, BW)], idxv.at[1])

    g0.wait()
    g1 = pltpu.make_async_copy(btab[1].at[idxv.at[1]], rowsB, sg1)
    g1.start()
    w0c = pltpu.make_async_copy(rowsA, out.at[0, pl.ds(base, BW)], sw0)
    w0c.start()
    pltpu.sync_copy(bidx[2].at[pl.ds(base, BW)], idxv.at[0])

    g1.wait()
    w0c.wait()
    g2 = pltpu.make_async_copy(btab[2].at[idxv.at[0]], rowsA, sg0)
    g2.start()
    w1c = pltpu.make_async_copy(rowsB, out.at[1, pl.ds(base, BW)], sw1)
    w1c.start()

    g2.wait()
    w2c = pltpu.make_async_copy(rowsA, out.at[2, pl.ds(base, BW)], sw0)
    w2c.start()
    w1c.wait()
    w2c.wait()

    lane = lax.broadcasted_iota(jnp.int32, (16,), 0)

    def make_idx_copies(s):
        Q = qidx[s]
        is_ag = (s == AG)

        def idx_copies(cc, pb):
            off = base * L + cc * CI
            cps = [pltpu.make_async_copy(
                Q.at[pl.ds(off + g * GI, GI)], gidx.at[pb * GN + g], qi[pb])
                for g in range(GN)]
            if is_ag:
                cps.append(pltpu.make_async_copy(
                    Q.at[pl.ds(off, CI)], gidx1[pb], qi[pb]))
            return cps
        return idx_copies

    all_idx_copies = [make_idx_copies(s) for s in range(N_SEQ)]

    for c in all_idx_copies[0](0, 0):
        c.start()

    for s in range(N_SEQ):
        fo = N_BIG + s
        is_ag = (s == AG)
        T = qtab[s]
        idx_copies = all_idx_copies[s]

        def gath_copies(pb):
            return [pltpu.make_async_copy(
                T.at[gidx.at[pb * GN + g]],
                seqbuf[pb].at[pl.ds(g * GI, GI)], qg[pb])
                for g in range(GN)]

        def wb_copy(cc, pb):
            return pltpu.make_async_copy(
                pool.at[pb], out.at[fo, pl.ds(base + cc * CR, CR)], qw[pb])

        def pool_chunk(cc, pb):
            buf = seqbuf[pb]

            def row_body(rr, _):
                if is_ag:
                    v1 = gidx1[pb][pl.ds(rr * L, 16)]
                    v2 = gidx1[pb][pl.ds(rr * L + 4, 16)]
                    n1 = plsc.all_reduce_population_count(v1 == 0)
                    n2 = plsc.all_reduce_population_count(
                        (v2 == 0) & (lane >= 12))
                    nz = n1 + n2
                    nzf = nz.astype(jnp.float32)
                    inv = 1.0 / jnp.maximum(L - nz, 1).astype(jnp.float32)
                for c in range(EMB // 16):
                    cs = pl.ds(c * 16, 16)
                    acc = buf[rr * L, cs]
                    for j in range(1, L):
                        acc = acc + buf[rr * L + j, cs]
                    if is_ag:
                        acc = (acc - nzf * w0v[0, cs]) * inv
                    else:
                        acc = acc * jnp.float32(1.0 / L)
                    pool[pb, rr, cs] = acc
                return 0
            lax.fori_loop(0, CR, row_body, 0)
            wb_copy(cc, pb).start()

        for c in idx_copies(0, 0):
            c.wait()
        for c in gath_copies(0):
            c.start()
        for c in idx_copies(1, 1):
            c.start()

        def chunk_pair(i, _):
            for pb in (0, 1):
                cc = i * 2 + pb

                @pl.when(cc < NCH - 1)
                def _():
                    for c in idx_copies(cc + 1, 1 - pb):
                        c.wait()
                    for c in gath_copies(1 - pb):
                        c.start()

                for c in gath_copies(pb):
                    c.wait()

                @pl.when(cc < NCH - 2)
                def _():
                    for c in idx_copies(cc + 2, pb):
                        c.start()

                if pb == 0 and s + 1 < N_SEQ:
                    @pl.when(cc == NCH - 2)
                    def _():
                        for c in all_idx_copies[s + 1](0, 0):
                            c.start()

                @pl.when(cc >= 2)
                def _():
                    wb_copy(cc - 2, pb).wait()

                pool_chunk(cc, pb)
            return 0
        lax.fori_loop(0, NCH // 2, chunk_pair, 0)
        wb_copy(NCH - 2, 0).wait()
        wb_copy(NCH - 1, 1).wait()


@jax.jit
def _sc_gather(*args):
    mesh = plsc.VectorSubcoreMesh(core_axis_name="c", subcore_axis_name="s")
    f = pl.kernel(
        _sc_body,
        out_type=jax.ShapeDtypeStruct((SC_FEAT, B, EMB), jnp.float32),
        mesh=mesh,
        scratch_types=[
            pltpu.VMEM((2, BW), jnp.int32),
            pltpu.VMEM((BW, EMB), jnp.float32),
            pltpu.VMEM((BW, EMB), jnp.float32),
            pltpu.VMEM((2 * GN, GI), jnp.int32),
            pltpu.VMEM((CI,), jnp.int32),
            pltpu.VMEM((CI,), jnp.int32),
            pltpu.VMEM((CI, EMB), jnp.float32),
            pltpu.VMEM((CI, EMB), jnp.float32),
            pltpu.VMEM((2, CR, EMB), jnp.float32),
            pltpu.VMEM((1, EMB), jnp.float32),
        ] + [pltpu.SemaphoreType.DMA] * 10,
        compiler_params=pltpu.CompilerParams(needs_layout_passes=False),
    )
    return f(*args)


SMALL_NS = [21, 21, 21, 21, 21, 21, 13, 21, 4, 21, 21, 21, 21, 21, 21, 7]


def _ctab_body(*refs):
    wd0_ref = refs[N_SMALL]
    out_ref = refs[N_SMALL + 1]
    for t, n in enumerate(SMALL_NS):
        ct = jnp.dot(refs[t][...],
                     wd0_ref[pl.ds((N_BIG + t) * EMB, EMB), :],
                     preferred_element_type=jnp.float32)
        out_ref[t] = jnp.concatenate(
            [ct, jnp.zeros((32 - n, 512), jnp.float32)], axis=0)


@jax.jit
def _tc_ctab(wd0, *stabs):
    return pl.pallas_call(
        _ctab_body,
        grid=(1,),
        in_specs=[pl.BlockSpec((n, EMB), lambda i: (0, 0))
                  for n in SMALL_NS]
        + [pl.BlockSpec((24 * EMB, 512), lambda i: (0, 0))],
        out_specs=pl.BlockSpec((N_SMALL, 32, 512), lambda i: (0, 0, 0)),
        out_shape=jax.ShapeDtypeStruct((N_SMALL, 32, 512), jnp.float32),
    )(*stabs, wd0)


def _mlp_body(x_ref, w0_ref, b0_ref, w1_ref, b1_ref, w2_ref, b2_ref,
              ccat_ref, *idx_refs):
    o_ref = idx_refs[-1]
    idx_refs = idx_refs[:-1]
    BM = o_ref.shape[0]
    acc = jnp.dot(x_ref[0], w0_ref[0], preferred_element_type=jnp.float32)
    for k in range(1, N_BIG):
        acc = acc + jnp.dot(x_ref[k], w0_ref[k],
                            preferred_element_type=jnp.float32)
    for k in range(N_SEQ):
        acc = acc + jnp.dot(x_ref[N_BIG + k],
                            w0_ref[N_BIG + N_SMALL + k],
                            preferred_element_type=jnp.float32)
    iota = lax.broadcasted_iota(jnp.int32, (BM, 32), 1)
    ohs = []
    for t in range(N_SMALL):
        idx = idx_refs[t][0, 0, :]
        ohs.append((idx[:, None] == iota).astype(jnp.float32))
    oh_cat = jnp.concatenate(ohs, axis=1)
    acc = acc + jnp.dot(oh_cat, ccat_ref[...],
                        preferred_element_type=jnp.float32)
    h = jnp.maximum(acc + b0_ref[...], 0.0)
    h = jnp.maximum(
        jnp.dot(h, w1_ref[...], preferred_element_type=jnp.float32)
        + b1_ref[...], 0.0)
    o_ref[...] = (jnp.dot(h, w2_ref[...], preferred_element_type=jnp.float32)
                  + b2_ref[...])


@jax.jit
def _tc_mlp(feats, w0r, b0, w1, b1, w2, b2, ccat, *sidx):
    BM = 512
    grid = (B // BM,)
    return pl.pallas_call(
        _mlp_body,
        grid=grid,
        in_specs=[
            pl.BlockSpec((SC_FEAT, BM, EMB), lambda i: (0, i, 0)),
            pl.BlockSpec((N_BIG + N_SMALL + N_SEQ, EMB, 512),
                         lambda i: (0, 0, 0)),
            pl.BlockSpec((1, 512), lambda i: (0, 0)),
            pl.BlockSpec((512, 256), lambda i: (0, 0)),
            pl.BlockSpec((1, 256), lambda i: (0, 0)),
            pl.BlockSpec((256, 128), lambda i: (0, 0)),
            pl.BlockSpec((1, 128), lambda i: (0, 0)),
            pl.BlockSpec((N_SMALL * 32, 512), lambda i: (0, 0)),
        ] + [pl.BlockSpec((1, 1, BM), lambda i: (i, 0, 0))] * N_SMALL,
        out_specs=pl.BlockSpec((BM, 128), lambda i: (i, 0)),
        out_shape=jax.ShapeDtypeStruct((B, 128), jnp.float32),
    )(feats, w0r, b0, w1, b1, w2, b2, ccat, *sidx)


def kernel(idx_track_uri, W_track_uri, idx_artist_uri, W_artist_uri,
           idx_album_uri, W_album_uri, idx_duration_ms, W_duration_ms,
           idx_track_pop, W_track_pop, idx_artist_pop, W_artist_pop,
           idx_artists_followers, W_artists_followers,
           idx_track_danceability, W_track_danceability,
           idx_track_energy, W_track_energy, idx_track_key, W_track_key,
           idx_track_loudness, W_track_loudness, idx_track_mode, W_track_mode,
           idx_track_speechiness, W_track_speechiness,
           idx_track_acousticness, W_track_acousticness,
           idx_track_instrumentalness, W_track_instrumentalness,
           idx_track_liveness, W_track_liveness,
           idx_track_valence, W_track_valence, idx_track_tempo, W_track_tempo,
           idx_time_signature, W_time_signature,
           idx_track_name, W_track_name, idx_artist_name, W_artist_name,
           idx_album_name, W_album_name, idx_artist_genres, W_artist_genres,
           idx_track_pl_titles, W_track_pl_titles,
           Wd0, bd0, Wd1, bd1, Wd2, bd2):
    big_tab = [W_track_uri, W_artist_uri, W_album_uri]
    big_idx = [idx_track_uri, idx_artist_uri, idx_album_uri]
    small_tab = [W_duration_ms, W_track_pop, W_artist_pop,
                 W_artists_followers, W_track_danceability, W_track_energy,
                 W_track_key, W_track_loudness, W_track_mode,
                 W_track_speechiness, W_track_acousticness,
                 W_track_instrumentalness, W_track_liveness, W_track_valence,
                 W_track_tempo, W_time_signature]
    small_idx = [idx_duration_ms, idx_track_pop, idx_artist_pop,
                 idx_artists_followers, idx_track_danceability,
                 idx_track_energy, idx_track_key, idx_track_loudness,
                 idx_track_mode, idx_track_speechiness,
                 idx_track_acousticness, idx_track_instrumentalness,
                 idx_track_liveness, idx_track_valence, idx_track_tempo,
                 idx_time_signature]
    seq_tab = [W_track_name, W_artist_name, W_album_name, W_artist_genres,
               W_track_pl_titles]
    seq_idx = [idx_track_name, idx_artist_name, idx_album_name,
               idx_artist_genres, idx_track_pl_titles]

    seq_flat = [jnp.reshape(ix, (B * L,)) for ix in seq_idx]

    feats = _sc_gather(*big_tab, *big_idx, *seq_tab, *seq_flat)

    ccat = jnp.reshape(_tc_ctab(Wd0, *small_tab), (N_SMALL * 32, 512))
    sidx3 = [jnp.reshape(ix, (B // 512, 1, 512)) for ix in small_idx]
    w0r = jnp.reshape(Wd0, (N_BIG + N_SMALL + N_SEQ, EMB, 512))
    return _tc_mlp(feats, w0r, jnp.reshape(bd0, (1, 512)),
                   Wd1, jnp.reshape(bd1, (1, 256)),
                   Wd2, jnp.reshape(bd2, (1, 128)), ccat, *sidx3)


# speedup vs baseline: 1.3201x; 1.3201x over previous
import functools

import jax
import jax.numpy as jnp
from jax import lax
from jax.experimental import pallas as pl
from jax.experimental.pallas import tpu as pltpu
from jax.experimental.pallas import tpu_sc as plsc

EMB = 128
B = 4096
L = 20
N_BIG = 3
N_SMALL = 16
N_SEQ = 5
AG = 3
SC_FEAT = N_BIG + N_SEQ

NC = 2
NS = 16
NW = NC * NS
BW = B // NW
CR = 16
NCH = BW // CR
CI = CR * L
GN = 4
GI = CI // GN


def _sc_body(*refs):
    btab = refs[0:N_BIG]
    bidx = refs[N_BIG:2 * N_BIG]
    qtab = refs[6:6 + N_SEQ]
    qidx = refs[11:11 + N_SEQ]
    out = refs[16]
    (idxv, rowsA, rowsB, gidx, seqA, seqB, pool, w0v,
     sg0, sg1, sw0, sw1, qi0, qi1, qg0, qg1, qw0, qw1) = refs[17:]
    qi = (qi0, qi1)
    qg = (qg0, qg1)
    qw = (qw0, qw1)
    seqbuf = (seqA, seqB)

    wid = lax.axis_index("s") * NC + lax.axis_index("c")
    base = wid * BW

    pltpu.sync_copy(refs[6 + AG].at[pl.ds(0, 1)], w0v)

    pltpu.sync_copy(bidx[0].at[pl.ds(base, BW)], idxv.at[0])
    g0 = pltpu.make_async_copy(btab[0].at[idxv.at[0]], rowsA, sg0)
    g0.start()
    pltpu.sync_copy(bidx[1].at[pl.ds(base, BW)], idxv.at[1])

    g0.wait()
    g1 = pltpu.make_async_copy(btab[1].at[idxv.at[1]], rowsB, sg1)
    g1.start()
    w0c = pltpu.make_async_copy(rowsA, out.at[0, pl.ds(base, BW)], sw0)
    w0c.start()
    pltpu.sync_copy(bidx[2].at[pl.ds(base, BW)], idxv.at[0])

    g1.wait()
    w0c.wait()
    g2 = pltpu.make_async_copy(btab[2].at[idxv.at[0]], rowsA, sg0)
    g2.start()
    w1c = pltpu.make_async_copy(rowsB, out.at[1, pl.ds(base, BW)], sw1)
    w1c.start()

    g2.wait()
    w2c = pltpu.make_async_copy(rowsA, out.at[2, pl.ds(base, BW)], sw0)
    w2c.start()
    w1c.wait()
    w2c.wait()

    lane = lax.broadcasted_iota(jnp.int32, (16,), 0)

    for s in range(N_SEQ):
        fo = N_BIG + s
        is_ag = (s == AG)
        Q = qidx[s]
        T = qtab[s]

        def idx_copies(cc, pb):
            row = base // GI * L + cc * GN
            return [pltpu.make_async_copy(
                Q.at[pl.ds(row, GN)], gidx.at[pl.ds(pb * GN, GN)], qi[pb])]

        def gath_copies(pb):
            return [pltpu.make_async_copy(
                T.at[gidx.at[pb * GN + g]],
                seqbuf[pb].at[pl.ds(g * GI, GI)], qg[pb])
                for g in range(GN)]

        def wb_copy(cc, pb):
            return pltpu.make_async_copy(
                pool.at[pb], out.at[fo, pl.ds(base + cc * CR, CR)], qw[pb])

        def pool_chunk(cc, pb):
            buf = seqbuf[pb]

            def row_body(rr, _):
                if is_ag:
                    g_row = pb * GN + rr // 4
                    g_col = (rr % 4) * L
                    v1 = gidx[g_row, pl.ds(g_col, 16)]
                    v2 = gidx[g_row, pl.ds(g_col + 4, 16)]
                    n1 = plsc.all_reduce_population_count(v1 == 0)
                    n2 = plsc.all_reduce_population_count(
                        (v2 == 0) & (lane >= 12))
                    nz = n1 + n2
                    nzf = nz.astype(jnp.float32)
                    inv = 1.0 / jnp.maximum(L - nz, 1).astype(jnp.float32)
                for c in range(EMB // 16):
                    cs = pl.ds(c * 16, 16)
                    terms = [buf[rr * L + j, cs] for j in range(L)]
                    while len(terms) > 1:
                        nxt = [terms[k] + terms[k + 1]
                               for k in range(0, len(terms) - 1, 2)]
                        if len(terms) % 2:
                            nxt.append(terms[-1])
                        terms = nxt
                    acc = terms[0]
                    if is_ag:
                        acc = (acc - nzf * w0v[0, cs]) * inv
                    else:
                        acc = acc * jnp.float32(1.0 / L)
                    pool[pb, rr, cs] = acc
                return 0
            lax.fori_loop(0, CR, row_body, 0)
            wb_copy(cc, pb).start()

        for c in idx_copies(0, 0):
            c.start()
        for c in idx_copies(0, 0):
            c.wait()
        for c in gath_copies(0):
            c.start()
        for c in idx_copies(1, 1):
            c.start()

        def chunk_pair(i, _):
            for pb in (0, 1):
                cc = i * 2 + pb
                for c in gath_copies(pb):
                    c.wait()

                @pl.when(cc < NCH - 1)
                def _():
                    for c in idx_copies(cc + 1, 1 - pb):
                        c.wait()
                    for c in gath_copies(1 - pb):
                        c.start()

                @pl.when(cc < NCH - 2)
                def _():
                    for c in idx_copies(cc + 2, pb):
                        c.start()

                @pl.when(cc >= 2)
                def _():
                    wb_copy(cc - 2, pb).wait()

                pool_chunk(cc, pb)
            return 0
        lax.fori_loop(0, NCH // 2, chunk_pair, 0)
        wb_copy(NCH - 2, 0).wait()
        wb_copy(NCH - 1, 1).wait()


@jax.jit
def _sc_gather(*args):
    mesh = plsc.VectorSubcoreMesh(core_axis_name="c", subcore_axis_name="s")
    f = pl.kernel(
        _sc_body,
        out_type=jax.ShapeDtypeStruct((SC_FEAT, B, EMB), jnp.float32),
        mesh=mesh,
        scratch_types=[
            pltpu.VMEM((2, BW), jnp.int32),
            pltpu.VMEM((BW, EMB), jnp.float32),
            pltpu.VMEM((BW, EMB), jnp.float32),
            pltpu.VMEM((2 * GN, GI), jnp.int32),
            pltpu.VMEM((CI, EMB), jnp.float32),
            pltpu.VMEM((CI, EMB), jnp.float32),
            pltpu.VMEM((2, CR, EMB), jnp.float32),
            pltpu.VMEM((1, EMB), jnp.float32),
        ] + [pltpu.SemaphoreType.DMA] * 10,
        compiler_params=pltpu.CompilerParams(needs_layout_passes=False,
                                             use_tc_tiling_on_sc=True),
    )
    return f(*args)


SMALL_NS = [21, 21, 21, 21, 21, 21, 13, 21, 4, 21, 21, 21, 21, 21, 21, 7]


def _ctab_body(*refs):
    wd0_ref = refs[N_SMALL]
    out_ref = refs[N_SMALL + 1]
    for t, n in enumerate(SMALL_NS):
        ct = jnp.dot(refs[t][...],
                     wd0_ref[pl.ds((N_BIG + t) * EMB, EMB), :],
                     preferred_element_type=jnp.float32)
        out_ref[t] = jnp.concatenate(
            [ct, jnp.zeros((32 - n, 512), jnp.float32)], axis=0)


@jax.jit
def _tc_ctab(wd0, *stabs):
    return pl.pallas_call(
        _ctab_body,
        grid=(1,),
        in_specs=[pl.BlockSpec((n, EMB), lambda i: (0, 0))
                  for n in SMALL_NS]
        + [pl.BlockSpec((24 * EMB, 512), lambda i: (0, 0))],
        out_specs=pl.BlockSpec((N_SMALL, 32, 512), lambda i: (0, 0, 0)),
        out_shape=jax.ShapeDtypeStruct((N_SMALL, 32, 512), jnp.float32),
    )(*stabs, wd0)


def _mlp_body(x_ref, w0_ref, b0_ref, w1_ref, b1_ref, w2_ref, b2_ref,
              ccat_ref, *idx_refs):
    o_ref = idx_refs[-1]
    idx_refs = idx_refs[:-1]
    BM = o_ref.shape[0]
    acc = jnp.dot(x_ref[0], w0_ref[0], preferred_element_type=jnp.float32)
    for k in range(1, N_BIG):
        acc = acc + jnp.dot(x_ref[k], w0_ref[k],
                            preferred_element_type=jnp.float32)
    for k in range(N_SEQ):
        acc = acc + jnp.dot(x_ref[N_BIG + k],
                            w0_ref[N_BIG + N_SMALL + k],
                            preferred_element_type=jnp.float32)
    iota = lax.broadcasted_iota(jnp.int32, (BM, 32), 1)
    ohs = []
    for t in range(N_SMALL):
        idx = idx_refs[t][0, 0, :]
        ohs.append((idx[:, None] == iota).astype(jnp.float32))
    oh_cat = jnp.concatenate(ohs, axis=1)
    acc = acc + jnp.dot(oh_cat, ccat_ref[...],
                        preferred_element_type=jnp.float32)
    h = jnp.maximum(acc + b0_ref[...], 0.0)
    h = jnp.maximum(
        jnp.dot(h, w1_ref[...], preferred_element_type=jnp.float32)
        + b1_ref[...], 0.0)
    o_ref[...] = (jnp.dot(h, w2_ref[...], preferred_element_type=jnp.float32)
                  + b2_ref[...])


@jax.jit
def _tc_mlp(feats, w0r, b0, w1, b1, w2, b2, ccat, *sidx):
    BM = 512
    grid = (B // BM,)
    return pl.pallas_call(
        _mlp_body,
        grid=grid,
        in_specs=[
            pl.BlockSpec((SC_FEAT, BM, EMB), lambda i: (0, i, 0)),
            pl.BlockSpec((N_BIG + N_SMALL + N_SEQ, EMB, 512),
                         lambda i: (0, 0, 0)),
            pl.BlockSpec((1, 512), lambda i: (0, 0)),
            pl.BlockSpec((512, 256), lambda i: (0, 0)),
            pl.BlockSpec((1, 256), lambda i: (0, 0)),
            pl.BlockSpec((256, 128), lambda i: (0, 0)),
            pl.BlockSpec((1, 128), lambda i: (0, 0)),
            pl.BlockSpec((N_SMALL * 32, 512), lambda i: (0, 0)),
        ] + [pl.BlockSpec((1, 1, BM), lambda i: (i, 0, 0))] * N_SMALL,
        out_specs=pl.BlockSpec((BM, 128), lambda i: (i, 0)),
        out_shape=jax.ShapeDtypeStruct((B, 128), jnp.float32),
    )(feats, w0r, b0, w1, b1, w2, b2, ccat, *sidx)


def kernel(idx_track_uri, W_track_uri, idx_artist_uri, W_artist_uri,
           idx_album_uri, W_album_uri, idx_duration_ms, W_duration_ms,
           idx_track_pop, W_track_pop, idx_artist_pop, W_artist_pop,
           idx_artists_followers, W_artists_followers,
           idx_track_danceability, W_track_danceability,
           idx_track_energy, W_track_energy, idx_track_key, W_track_key,
           idx_track_loudness, W_track_loudness, idx_track_mode, W_track_mode,
           idx_track_speechiness, W_track_speechiness,
           idx_track_acousticness, W_track_acousticness,
           idx_track_instrumentalness, W_track_instrumentalness,
           idx_track_liveness, W_track_liveness,
           idx_track_valence, W_track_valence, idx_track_tempo, W_track_tempo,
           idx_time_signature, W_time_signature,
           idx_track_name, W_track_name, idx_artist_name, W_artist_name,
           idx_album_name, W_album_name, idx_artist_genres, W_artist_genres,
           idx_track_pl_titles, W_track_pl_titles,
           Wd0, bd0, Wd1, bd1, Wd2, bd2):
    big_tab = [W_track_uri, W_artist_uri, W_album_uri]
    big_idx = [idx_track_uri, idx_artist_uri, idx_album_uri]
    small_tab = [W_duration_ms, W_track_pop, W_artist_pop,
                 W_artists_followers, W_track_danceability, W_track_energy,
                 W_track_key, W_track_loudness, W_track_mode,
                 W_track_speechiness, W_track_acousticness,
                 W_track_instrumentalness, W_track_liveness, W_track_valence,
                 W_track_tempo, W_time_signature]
    small_idx = [idx_duration_ms, idx_track_pop, idx_artist_pop,
                 idx_artists_followers, idx_track_danceability,
                 idx_track_energy, idx_track_key, idx_track_loudness,
                 idx_track_mode, idx_track_speechiness,
                 idx_track_acousticness, idx_track_instrumentalness,
                 idx_track_liveness, idx_track_valence, idx_track_tempo,
                 idx_time_signature]
    seq_tab = [W_track_name, W_artist_name, W_album_name, W_artist_genres,
               W_track_pl_titles]
    seq_idx = [idx_track_name, idx_artist_name, idx_album_name,
               idx_artist_genres, idx_track_pl_titles]

    seq_flat = [jnp.reshape(ix, (B * L // GI, GI)) for ix in seq_idx]

    feats = _sc_gather(*big_tab, *big_idx, *seq_tab, *seq_flat)

    ccat = jnp.reshape(_tc_ctab(Wd0, *small_tab), (N_SMALL * 32, 512))
    sidx3 = [jnp.reshape(ix, (B // 512, 1, 512)) for ix in small_idx]
    w0r = jnp.reshape(Wd0, (N_BIG + N_SMALL + N_SEQ, EMB, 512))
    return _tc_mlp(feats, w0r, jnp.reshape(bd0, (1, 512)),
                   Wd1, jnp.reshape(bd1, (1, 256)),
                   Wd2, jnp.reshape(bd2, (1, 128)), ccat, *sidx3)
